# baseline (speedup 1.0000x reference)
import functools

import jax
import jax.numpy as jnp
from jax import lax
from jax.experimental import pallas as pl
from jax.experimental.pallas import tpu as pltpu
from jax.experimental.pallas import tpu_sc as plsc

BATCH = 4096
HIST = 200
D = 64

NC, NS = 2, 16
NW = NC * NS
ROWS = BATCH * HIST
G = 128
GROUPS = ROWS // G
GPW = GROUPS // NW
CG = 4
CHUNKS = GPW // CG

_mesh = plsc.VectorSubcoreMesh(core_axis_name="c", subcore_axis_name="s")


@functools.partial(
    pl.kernel,
    out_type=jax.ShapeDtypeStruct((ROWS, D), jnp.float32),
    mesh=_mesh,
    scratch_types=[
        pltpu.VMEM((CG, G), jnp.int32),
        pltpu.VMEM((CG * G, D), jnp.float32),
        pltpu.SemaphoreType.DMA,
    ],
    compiler_params=pltpu.CompilerParams(use_tc_tiling_on_sc=False),
)
def _gather_kernel(idx_hbm, table_hbm, out_hbm, idx_v, rows_v, sem):
    wid = lax.axis_index("s") * NC + lax.axis_index("c")

    @pl.loop(0, CHUNKS)
    def _chunk(c):
        gbase = wid * GPW + c * CG
        pltpu.sync_copy(idx_hbm.at[pl.ds(gbase, CG)], idx_v)
        copies = [
            pltpu.async_copy(
                table_hbm.at[idx_v.at[j]],
                rows_v.at[pl.ds(j * G, G)],
                sem,
            )
            for j in range(CG)
        ]
        for cp in copies:
            cp.wait()
        pltpu.sync_copy(rows_v, out_hbm.at[pl.ds(gbase * G, CG * G)])


def kernel(x, table):
    idx = x.reshape(GROUPS, G).astype(jnp.int32)
    out = _gather_kernel(idx, table)
    return out.reshape(BATCH, HIST, D)


# speedup vs baseline: 1.0430x; 1.0430x over previous
import functools

import jax
import jax.numpy as jnp
from jax import lax
from jax.experimental import pallas as pl
from jax.experimental.pallas import tpu as pltpu
from jax.experimental.pallas import tpu_sc as plsc

BATCH = 4096
HIST = 200
D = 64

NC, NS = 2, 16
NW = NC * NS
ROWS = BATCH * HIST
G = 128
GROUPS = ROWS // G
GPW = GROUPS // NW
CG = 4
CHUNKS = GPW // CG
CROWS = CG * G

_mesh = plsc.VectorSubcoreMesh(core_axis_name="c", subcore_axis_name="s")


@functools.partial(
    pl.kernel,
    out_type=jax.ShapeDtypeStruct((ROWS, D), jnp.float32),
    mesh=_mesh,
    scratch_types=[
        pltpu.VMEM((GPW, G), jnp.int32),
        pltpu.VMEM((CROWS, D), jnp.float32),
        pltpu.VMEM((CROWS, D), jnp.float32),
        pltpu.SemaphoreType.DMA,
        pltpu.SemaphoreType.DMA,
    ],
    compiler_params=pltpu.CompilerParams(use_tc_tiling_on_sc=False),
)
def _gather_kernel(idx_hbm, table_hbm, out_hbm, idx_all, rows0, rows1,
                   sem0, sem1):
    wid = lax.axis_index("s") * NC + lax.axis_index("c")
    gbase = wid * GPW

    def fire(c, rows_buf, sem):
        for j in range(CG):
            pltpu.async_copy(
                table_hbm.at[idx_all.at[c * CG + j]],
                rows_buf.at[pl.ds(j * G, G)],
                sem,
            )

    def drain_and_writeback(c, rows_buf, sem):
        pltpu.make_async_copy(
            table_hbm.at[pl.ds(0, CROWS)], rows_buf, sem).wait()
        pltpu.sync_copy(
            rows_buf, out_hbm.at[pl.ds((gbase + c * CG) * G, CROWS)])

    pltpu.sync_copy(idx_hbm.at[pl.ds(gbase, GPW)], idx_all)
    fire(0, rows0, sem0)

    @pl.loop(0, CHUNKS - 2, step=2)
    def _pair(cc):
        fire(cc + 1, rows1, sem1)
        drain_and_writeback(cc, rows0, sem0)
        fire(cc + 2, rows0, sem0)
        drain_and_writeback(cc + 1, rows1, sem1)

    fire(CHUNKS - 1, rows1, sem1)
    drain_and_writeback(CHUNKS - 2, rows0, sem0)
    drain_and_writeback(CHUNKS - 1, rows1, sem1)


def kernel(x, table):
    idx = x.reshape(GROUPS, G).astype(jnp.int32)
    out = _gather_kernel(idx, table)
    return out.reshape(BATCH, HIST, D)
